# baseline (speedup 1.0000x reference)
import jax
import jax.numpy as jnp
from jax.experimental import pallas as pl
from jax.experimental.pallas import tpu as pltpu

EMB = 128
NE = 256
NF = 1024
TOPK = 10
BIG = 1e30
GB = 8
FC = 256
NC = NF // FC


def _dot_ct(a, b):
    return jax.lax.dot_general(
        a, b, (((1,), (1,)), ((), ())), preferred_element_type=jnp.float32)


def _dot_00(a, b):
    return jax.lax.dot_general(
        a, b, (((0,), (0,)), ((), ())), preferred_element_type=jnp.float32)


def _qnorm(v):
    return jnp.sum(v * v, axis=1, keepdims=True)


def _hoppy_body(nb_ref, rel_ref, a1_ref, a2_ref, w0_ref, b0_ref, w1_ref,
                b1_ref, fr_ref, fa1_ref, fa2_ref, ent_ref, s0_ref, res_ref,
                acc_ref):
    g = pl.program_id(0)
    c = pl.program_id(1)
    ones_row = jnp.ones((1, EMB), dtype=jnp.float32)
    fiota = jax.lax.broadcasted_iota(jnp.int32, (1, FC), 1) + c * FC

    m1_cols, m2_cols, m0_cols = [], [], []
    for i in range(GB):
        nb = nb_ref[g * GB + i]
        rel = rel_ref[i]
        a1 = a1_ref[i]
        a2 = a2_ref[i]
        fr = fr_ref[i]
        fa1 = fa1_ref[i]
        fa2 = fa2_ref[i]
        ent = ent_ref[i]

        r1 = jax.lax.dot_general(
            rel, w0_ref[...], (((1,), (0,)), ((), ())),
            preferred_element_type=jnp.float32) + b0_ref[...]
        r2 = jax.lax.dot_general(
            rel, w1_ref[...], (((1,), (0,)), ((), ())),
            preferred_element_type=jnp.float32) + b1_ref[...]

        sq_f = fr * fr + fa1 * fa1 + fa2 * fa2
        n0 = _dot_ct(ones_row, sq_f)
        qrows = jnp.concatenate([rel, r1, r2], axis=0)
        dfr = _dot_ct(qrows, fr)
        dfa1 = _dot_ct(a1, fa1)
        dfa2 = _dot_ct(a2, fa2)

        qn_rel, qn_r1, qn_r2 = _qnorm(rel), _qnorm(r1), _qnorm(r2)
        qn_a1, qn_a2 = _qnorm(a1), _qnorm(a2)

        valid = fiota < nb
        c0 = jnp.where(
            valid,
            n0 - 2.0 * (dfr[0:1] + dfa1 + dfa2) + (qn_rel + qn_a1 + qn_a2),
            BIG)
        c1p = jnp.where(
            valid, n0 - 2.0 * (dfr[1:2] + dfa1) + (qn_r1 + qn_a1), BIG)
        c2p = jnp.where(
            valid, n0 - 2.0 * (dfr[2:3] + dfa2) + (qn_r2 + qn_a2), BIG)

        e2 = _dot_ct(ent * ent, ones_row)
        entm2 = ent * -2.0
        g1 = _dot_ct(entm2, fa2)
        m1 = jnp.min(c1p + g1, axis=1, keepdims=True)
        g2 = _dot_ct(entm2, fa1)
        m2 = jnp.min(c2p + g2, axis=1, keepdims=True)
        m0 = jnp.min(c0, axis=(0, 1), keepdims=True)

        m1_cols.append(m1 + e2)
        m2_cols.append(m2 + e2)
        m0_cols.append(jnp.broadcast_to(m0, (NE, 1)))

    m_new = jnp.concatenate(m1_cols + m2_cols + m0_cols, axis=1)

    @pl.when(c == 0)
    def _():
        acc_ref[...] = m_new

    @pl.when(c > 0)
    def _():
        acc_ref[...] = jnp.minimum(acc_ref[...], m_new)

    @pl.when(c == NC - 1)
    def _():
        eye_ne = jnp.eye(NE, dtype=jnp.float32)
        eiota = jax.lax.broadcasted_iota(jnp.int32, (GB, NE), 1)
        m_rows = _dot_00(acc_ref[...], eye_ne)
        sc_rows = jnp.exp(-0.5 * jnp.maximum(m_rows, 0.0))
        ns = sc_rows[0:GB]
        z2 = sc_rows[GB:2 * GB]
        s0r = sc_rows[2 * GB:]

        vals = ns
        member = jnp.zeros(ns.shape, dtype=jnp.bool_)
        for _ in range(TOPK):
            m = jnp.max(vals, axis=1, keepdims=True)
            idx = jnp.min(jnp.where(vals == m, eiota, NE),
                          axis=1, keepdims=True)
            onehot = (eiota == idx)
            member = jnp.logical_or(member, onehot)
            vals = jnp.where(onehot, -1.0, vals)

        combined = jnp.where(member, jnp.minimum(z2, ns), -1.0)
        res = jnp.max(combined, axis=1, keepdims=True)
        s0v = jnp.max(s0r, axis=1, keepdims=True)
        for i in range(GB):
            s0_ref[i] = jnp.broadcast_to(s0v[i:i + 1], (1, EMB))
            res_ref[i] = jnp.broadcast_to(res[i:i + 1], (1, EMB))


@jax.jit
def _run(rel, arg1, arg2, fact_rel, fact_arg1, fact_arg2, nb_facts,
         entity_embeddings, W0, b0, W1, b1):
    Bb = rel.shape[0]
    grid = (Bb // GB, NC)
    vec_spec = pl.BlockSpec((GB, 1, EMB), lambda b, c: (b, 0, 0))
    mat_spec = pl.BlockSpec((EMB, EMB), lambda b, c: (0, 0))
    row_spec = pl.BlockSpec((1, EMB), lambda b, c: (0, 0))
    fact_spec = pl.BlockSpec((GB, FC, EMB), lambda b, c: (b, c, 0))
    ent_spec = pl.BlockSpec((GB, NE, EMB), lambda b, c: (b, 0, 0))
    out_spec = pl.BlockSpec((GB, 1, EMB), lambda b, c: (b, 0, 0))

    s0, res = pl.pallas_call(
        _hoppy_body,
        grid=grid,
        in_specs=[
            pl.BlockSpec(memory_space=pltpu.SMEM),
            vec_spec, vec_spec, vec_spec,
            mat_spec, row_spec, mat_spec, row_spec,
            fact_spec, fact_spec, fact_spec,
            ent_spec,
        ],
        out_specs=[out_spec, out_spec],
        out_shape=[
            jax.ShapeDtypeStruct((Bb, 1, EMB), jnp.float32),
            jax.ShapeDtypeStruct((Bb, 1, EMB), jnp.float32),
        ],
        scratch_shapes=[pltpu.VMEM((NE, 3 * GB), jnp.float32)],
        compiler_params=pltpu.CompilerParams(
            dimension_semantics=("arbitrary", "arbitrary")),
    )(nb_facts, rel.reshape(Bb, 1, EMB), arg1.reshape(Bb, 1, EMB),
      arg2.reshape(Bb, 1, EMB), W0, b0.reshape(1, EMB), W1,
      b1.reshape(1, EMB), fact_rel, fact_arg1, fact_arg2, entity_embeddings)
    return s0[:, 0, 0], res[:, 0, 0]


def kernel(rel, arg1, arg2, fact_rel, fact_arg1, fact_arg2, nb_facts,
           entity_embeddings, nb_entities, W0, b0, W1, b1, depth):
    s0, res = _run(rel, arg1, arg2, fact_rel, fact_arg1, fact_arg2,
                   nb_facts, entity_embeddings, W0, b0, W1, b1)
    return jnp.where(depth <= 0, s0, jnp.maximum(s0, res))


# speedup vs baseline: 1.3842x; 1.3842x over previous
import functools

import jax
import jax.numpy as jnp
from jax.experimental import pallas as pl
from jax.experimental.pallas import tpu as pltpu

EMB = 128
NE = 256
NF = 1024
TOPK = 10
BIG = 1e30
GB = 8


def _dot_ct(a, b):
    return jax.lax.dot_general(
        a, b, (((1,), (1,)), ((), ())), preferred_element_type=jnp.float32)


def _dot_00(a, b):
    return jax.lax.dot_general(
        a, b, (((0,), (0,)), ((), ())), preferred_element_type=jnp.float32)


def _qnorm(v):
    return jnp.sum(v * v, axis=1, keepdims=True)


def _hoppy_body(nb_ref, rel_ref, a1_ref, a2_ref, w0_ref, b0_ref, w1_ref,
                b1_ref, fr_ref, fa1_ref, fa2_ref, ent_ref, s0_ref, res_ref):
    g = pl.program_id(0)
    ones_row = jnp.ones((1, EMB), dtype=jnp.float32)
    eye_ne = jnp.eye(NE, dtype=jnp.float32)
    fiota = jax.lax.broadcasted_iota(jnp.int32, (1, NF), 1)
    eiota = jax.lax.broadcasted_iota(jnp.int32, (GB, NE), 1)

    m_cols = []
    for i in range(GB):
        nb = nb_ref[g * GB + i]
        rel = rel_ref[i]
        a1 = a1_ref[i]
        a2 = a2_ref[i]
        fr = fr_ref[i]
        fa1 = fa1_ref[i]
        fa2 = fa2_ref[i]
        ent = ent_ref[i]

        r1 = jax.lax.dot_general(
            rel, w0_ref[...], (((1,), (0,)), ((), ())),
            preferred_element_type=jnp.float32) + b0_ref[...]
        r2 = jax.lax.dot_general(
            rel, w1_ref[...], (((1,), (0,)), ((), ())),
            preferred_element_type=jnp.float32) + b1_ref[...]

        sq_f = fr * fr + fa1 * fa1 + fa2 * fa2
        n0 = _dot_ct(ones_row, sq_f)
        qrows = jnp.concatenate([rel, r1, r2], axis=0)
        dfr = _dot_ct(qrows, fr)
        dfa1 = _dot_ct(a1, fa1)
        dfa2 = _dot_ct(a2, fa2)

        qn_rel, qn_r1, qn_r2 = _qnorm(rel), _qnorm(r1), _qnorm(r2)
        qn_a1, qn_a2 = _qnorm(a1), _qnorm(a2)

        valid = fiota < nb
        c0 = jnp.where(
            valid,
            n0 - 2.0 * (dfr[0:1] + dfa1 + dfa2) + (qn_rel + qn_a1 + qn_a2),
            BIG)
        c1p = jnp.where(
            valid, n0 - 2.0 * (dfr[1:2] + dfa1) + (qn_r1 + qn_a1), BIG)
        c2p = jnp.where(
            valid, n0 - 2.0 * (dfr[2:3] + dfa2) + (qn_r2 + qn_a2), BIG)

        m0 = jnp.min(c0, axis=(0, 1), keepdims=True)
        s0 = jnp.exp(-0.5 * jnp.maximum(m0, 0.0))
        s0_ref[i] = jnp.broadcast_to(s0, (1, EMB))

        e2 = _dot_ct(ent * ent, ones_row)
        entm2 = ent * -2.0
        g1 = _dot_ct(entm2, fa2)
        m1 = jnp.min(c1p + g1, axis=1, keepdims=True)
        g2 = _dot_ct(entm2, fa1)
        m2 = jnp.min(c2p + g2, axis=1, keepdims=True)
        m_cols.insert(i, m1 + e2)
        m_cols.append(m2 + e2)

    m_all = jnp.concatenate(m_cols, axis=1)
    m_rows = _dot_00(m_all, eye_ne)
    sc_rows = jnp.exp(-0.5 * jnp.maximum(m_rows, 0.0))
    ns = sc_rows[:GB]
    z2 = sc_rows[GB:]

    vals = ns
    member = jnp.zeros(ns.shape, dtype=jnp.bool_)
    for _ in range(TOPK):
        m = jnp.max(vals, axis=1, keepdims=True)
        idx = jnp.min(jnp.where(vals == m, eiota, NE),
                      axis=1, keepdims=True)
        onehot = (eiota == idx)
        member = jnp.logical_or(member, onehot)
        vals = jnp.where(onehot, -1.0, vals)

    combined = jnp.where(member, jnp.minimum(z2, ns), -1.0)
    res = jnp.max(combined, axis=1, keepdims=True)
    for i in range(GB):
        res_ref[i] = jnp.broadcast_to(res[i:i + 1], (1, EMB))


@jax.jit
def _run(rel, arg1, arg2, fact_rel, fact_arg1, fact_arg2, nb_facts,
         entity_embeddings, W0, b0, W1, b1):
    Bb = rel.shape[0]
    grid = (Bb // GB,)
    vec_spec = pl.BlockSpec((GB, 1, EMB), lambda b: (b, 0, 0))
    mat_spec = pl.BlockSpec((EMB, EMB), lambda b: (0, 0))
    row_spec = pl.BlockSpec((1, EMB), lambda b: (0, 0))
    fact_spec = pl.BlockSpec((GB, NF, EMB), lambda b: (b, 0, 0))
    ent_spec = pl.BlockSpec((GB, NE, EMB), lambda b: (b, 0, 0))
    out_spec = pl.BlockSpec((GB, 1, EMB), lambda b: (b, 0, 0))

    s0, res = pl.pallas_call(
        _hoppy_body,
        grid=grid,
        in_specs=[
            pl.BlockSpec(memory_space=pltpu.SMEM),
            vec_spec, vec_spec, vec_spec,
            mat_spec, row_spec, mat_spec, row_spec,
            fact_spec, fact_spec, fact_spec,
            ent_spec,
        ],
        out_specs=[out_spec, out_spec],
        out_shape=[
            jax.ShapeDtypeStruct((Bb, 1, EMB), jnp.float32),
            jax.ShapeDtypeStruct((Bb, 1, EMB), jnp.float32),
        ],
        compiler_params=pltpu.CompilerParams(
            dimension_semantics=("arbitrary",)),
    )(nb_facts, rel.reshape(Bb, 1, EMB), arg1.reshape(Bb, 1, EMB),
      arg2.reshape(Bb, 1, EMB), W0, b0.reshape(1, EMB), W1,
      b1.reshape(1, EMB), fact_rel, fact_arg1, fact_arg2, entity_embeddings)
    return s0[:, 0, 0], res[:, 0, 0]


def kernel(rel, arg1, arg2, fact_rel, fact_arg1, fact_arg2, nb_facts,
           entity_embeddings, nb_entities, W0, b0, W1, b1, depth):
    s0, res = _run(rel, arg1, arg2, fact_rel, fact_arg1, fact_arg2,
                   nb_facts, entity_embeddings, W0, b0, W1, b1)
    return jnp.where(depth <= 0, s0, jnp.maximum(s0, res))


# speedup vs baseline: 1.3844x; 1.0001x over previous
import jax
import jax.numpy as jnp
from jax.experimental import pallas as pl
from jax.experimental.pallas import tpu as pltpu

EMB = 128
NE = 256
NF = 1024
TOPK = 10
BIG = 1e30
GB = 8


def _dot_ct(a, b):
    return jax.lax.dot_general(
        a, b, (((1,), (1,)), ((), ())), preferred_element_type=jnp.float32)


def _dot_00(a, b):
    return jax.lax.dot_general(
        a, b, (((0,), (0,)), ((), ())), preferred_element_type=jnp.float32)


def _qnorm(v):
    return jnp.sum(v * v, axis=1, keepdims=True)


def _hoppy_body(nb_ref, rel_ref, a1_ref, a2_ref, w0_ref, b0_ref, w1_ref,
                b1_ref, fr_ref, fa1_ref, fa2_ref, ent_ref, s0_ref, res_ref):
    g = pl.program_id(0)
    ones_row = jnp.ones((1, EMB), dtype=jnp.float32)
    eye_ne = jnp.eye(NE, dtype=jnp.float32)
    fiota = jax.lax.broadcasted_iota(jnp.int32, (1, NF), 1)
    eiota = jax.lax.broadcasted_iota(jnp.int32, (GB, NE), 1)

    m_cols = []
    for i in range(GB):
        nb = nb_ref[g * GB + i]
        rel = rel_ref[i]
        a1 = a1_ref[i]
        a2 = a2_ref[i]
        fr = fr_ref[i]
        fa1 = fa1_ref[i]
        fa2 = fa2_ref[i]
        ent = ent_ref[i]

        r1 = jax.lax.dot_general(
            rel, w0_ref[...], (((1,), (0,)), ((), ())),
            preferred_element_type=jnp.float32) + b0_ref[...]
        r2 = jax.lax.dot_general(
            rel, w1_ref[...], (((1,), (0,)), ((), ())),
            preferred_element_type=jnp.float32) + b1_ref[...]

        sq_f = fr * fr + fa1 * fa1 + fa2 * fa2
        n0 = _dot_ct(ones_row, sq_f)
        qrows = jnp.concatenate([rel, r1, r2], axis=0)
        dfr = _dot_ct(qrows, fr)
        dfa1 = _dot_ct(a1, fa1)
        dfa2 = _dot_ct(a2, fa2)

        qn_rel, qn_r1, qn_r2 = _qnorm(rel), _qnorm(r1), _qnorm(r2)
        qn_a1, qn_a2 = _qnorm(a1), _qnorm(a2)

        valid = fiota < nb
        c0 = jnp.where(
            valid,
            n0 - 2.0 * (dfr[0:1] + dfa1 + dfa2) + (qn_rel + qn_a1 + qn_a2),
            BIG)
        c1p = jnp.where(
            valid, n0 - 2.0 * (dfr[1:2] + dfa1) + (qn_r1 + qn_a1), BIG)
        c2p = jnp.where(
            valid, n0 - 2.0 * (dfr[2:3] + dfa2) + (qn_r2 + qn_a2), BIG)

        m0 = jnp.min(c0, axis=(0, 1), keepdims=True)
        s0 = jnp.exp(-0.5 * jnp.maximum(m0, 0.0))
        s0_ref[i] = jnp.broadcast_to(s0, (1, EMB))

        e2 = _dot_ct(ent * ent, ones_row)
        entm2 = ent * -2.0
        g1 = _dot_ct(entm2, fa2)
        m1 = jnp.min(c1p + g1, axis=1, keepdims=True)
        g2 = _dot_ct(entm2, fa1)
        m2 = jnp.min(c2p + g2, axis=1, keepdims=True)
        m_cols.insert(i, m1 + e2)
        m_cols.append(m2 + e2)

    m_all = jnp.concatenate(m_cols, axis=1)
    m_rows = _dot_00(m_all, eye_ne)
    sc_rows = jnp.exp(-0.5 * jnp.maximum(m_rows, 0.0))
    ns = sc_rows[:GB]
    z2 = sc_rows[GB:]

    vals = ns
    member = jnp.zeros(ns.shape, dtype=jnp.bool_)
    for _ in range(TOPK):
        m = jnp.max(vals, axis=1, keepdims=True)
        idx = jnp.min(jnp.where(vals == m, eiota, NE),
                      axis=1, keepdims=True)
        onehot = (eiota == idx)
        member = jnp.logical_or(member, onehot)
        vals = jnp.where(onehot, -1.0, vals)

    combined = jnp.where(member, jnp.minimum(z2, ns), -1.0)
    res = jnp.max(combined, axis=1, keepdims=True)
    for i in range(GB):
        res_ref[i] = jnp.broadcast_to(res[i:i + 1], (1, EMB))


@jax.jit
def _run(rel, arg1, arg2, fact_rel, fact_arg1, fact_arg2, nb_facts,
         entity_embeddings, W0, b0, W1, b1):
    Bb = rel.shape[0]
    grid = (Bb // GB,)
    vec_spec = pl.BlockSpec((GB, 1, EMB), lambda b: (b, 0, 0))
    mat_spec = pl.BlockSpec((EMB, EMB), lambda b: (0, 0))
    row_spec = pl.BlockSpec((1, EMB), lambda b: (0, 0))
    fact_spec = pl.BlockSpec((GB, NF, EMB), lambda b: (b, 0, 0))
    ent_spec = pl.BlockSpec((GB, NE, EMB), lambda b: (b, 0, 0))
    out_spec = pl.BlockSpec((GB, 1, EMB), lambda b: (b, 0, 0))

    s0, res = pl.pallas_call(
        _hoppy_body,
        grid=grid,
        in_specs=[
            pl.BlockSpec(memory_space=pltpu.SMEM),
            vec_spec, vec_spec, vec_spec,
            mat_spec, row_spec, mat_spec, row_spec,
            fact_spec, fact_spec, fact_spec,
            ent_spec,
        ],
        out_specs=[out_spec, out_spec],
        out_shape=[
            jax.ShapeDtypeStruct((Bb, 1, EMB), jnp.float32),
            jax.ShapeDtypeStruct((Bb, 1, EMB), jnp.float32),
        ],
        compiler_params=pltpu.CompilerParams(
            dimension_semantics=("arbitrary",)),
    )(nb_facts, rel.reshape(Bb, 1, EMB), arg1.reshape(Bb, 1, EMB),
      arg2.reshape(Bb, 1, EMB), W0, b0.reshape(1, EMB), W1,
      b1.reshape(1, EMB), fact_rel, fact_arg1, fact_arg2, entity_embeddings)
    return s0[:, 0, 0], res[:, 0, 0]


def kernel(rel, arg1, arg2, fact_rel, fact_arg1, fact_arg2, nb_facts,
           entity_embeddings, nb_entities, W0, b0, W1, b1, depth):
    s0, res = _run(rel, arg1, arg2, fact_rel, fact_arg1, fact_arg2,
                   nb_facts, entity_embeddings, W0, b0, W1, b1)
    return jnp.where(depth <= 0, s0, jnp.maximum(s0, res))
